# baseline (speedup 1.0000x reference)
import functools

import jax
import jax.numpy as jnp
from jax import lax
from jax.experimental import pallas as pl
from jax.experimental.pallas import tpu as pltpu
from jax.experimental.pallas import tpu_sc as plsc

_CHUNK = 16
_NBUF_T = 4
_NBUF_S = 2
_RA_T = 2
_RA_S = 1
_ROWS_T = 128


def _pipe(start_read, start_write, n_chunks, nbuf, read_ahead):
    reads = [None] * n_chunks
    writes = [None] * n_chunks
    for i in range(min(read_ahead, n_chunks)):
        reads[i] = start_read(i)
    for i in range(n_chunks):
        nxt = i + read_ahead
        if nxt < n_chunks:
            if nxt >= nbuf:
                writes[nxt - nbuf].wait()
            reads[nxt] = start_read(nxt)
        reads[i].wait()
        writes[i] = start_write(i)
        yield
    for i in range(max(0, n_chunks - nbuf), n_chunks):
        writes[i].wait()
    yield


@functools.lru_cache(maxsize=None)
def _make_copy(n_seq, d_emb):
    info = plsc.get_sparse_core_info()
    nc, ns = info.num_cores, info.num_subcores
    nw = nc * ns
    rows_per_w = n_seq // nw
    rows_t = min(_ROWS_T, rows_per_w)
    rows_s = rows_per_w - rows_t
    n_ct = rows_t // _CHUNK
    n_cs = rows_s // _CHUNK

    mesh = plsc.VectorSubcoreMesh(core_axis_name="c", subcore_axis_name="s")

    @functools.partial(
        pl.kernel,
        mesh=mesh,
        out_type=jax.ShapeDtypeStruct((n_seq, d_emb), jnp.float32),
        scratch_types=[
            pltpu.VMEM((_NBUF_T, _CHUNK, d_emb), jnp.float32),
            pltpu.VMEM_SHARED((ns, _NBUF_S, _CHUNK, d_emb), jnp.float32),
            pltpu.SemaphoreType.DMA((_NBUF_T,)),
            pltpu.SemaphoreType.DMA((_NBUF_T,)),
            pltpu.SemaphoreType.DMA((_NBUF_S,)),
            pltpu.SemaphoreType.DMA((_NBUF_S,)),
        ],
    )
    def copy_kernel(table_hbm, out_hbm, tbufs, sbufs, trs, tws, srs, sws):
        sid = lax.axis_index("s")
        wid = sid * nc + lax.axis_index("c")
        base = wid * rows_per_w
        base_s = base + rows_t

        def t_read(i):
            return pltpu.async_copy(
                table_hbm.at[pl.ds(base + i * _CHUNK, _CHUNK)],
                tbufs.at[i % _NBUF_T],
                trs.at[i % _NBUF_T],
            )

        def t_write(i):
            return pltpu.async_copy(
                tbufs.at[i % _NBUF_T],
                out_hbm.at[pl.ds(base + i * _CHUNK, _CHUNK)],
                tws.at[i % _NBUF_T],
            )

        def s_read(i):
            return pltpu.async_copy(
                table_hbm.at[pl.ds(base_s + i * _CHUNK, _CHUNK)],
                sbufs.at[sid, i % _NBUF_S],
                srs.at[i % _NBUF_S],
            )

        def s_write(i):
            return pltpu.async_copy(
                sbufs.at[sid, i % _NBUF_S],
                out_hbm.at[pl.ds(base_s + i * _CHUNK, _CHUNK)],
                sws.at[i % _NBUF_S],
            )

        pipes = []
        if n_ct:
            pipes.append(_pipe(t_read, t_write, n_ct, _NBUF_T, _RA_T))
        if n_cs:
            pipes.append(_pipe(s_read, s_write, n_cs, _NBUF_S, _RA_S))
        alive = list(pipes)
        while alive:
            for g in list(alive):
                try:
                    next(g)
                except StopIteration:
                    alive.remove(g)

    return copy_kernel


def kernel(x, table):
    n_seq = x.shape[-1]
    return _make_copy(n_seq, table.shape[1])(table)


# speedup vs baseline: 1.0207x; 1.0207x over previous
import functools

import jax
import jax.numpy as jnp
from jax import lax
from jax.experimental import pallas as pl
from jax.experimental.pallas import tpu as pltpu
from jax.experimental.pallas import tpu_sc as plsc

_NBUF = 6
_CHUNK_ROWS = 16
_READ_AHEAD = 3


@functools.lru_cache(maxsize=None)
def _make_copy(n_seq, d_emb):
    info = plsc.get_sparse_core_info()
    nc, ns = info.num_cores, info.num_subcores
    nw = nc * ns
    rows_per_w = n_seq // nw
    n_chunks = rows_per_w // _CHUNK_ROWS

    mesh = plsc.VectorSubcoreMesh(core_axis_name="c", subcore_axis_name="s")

    @functools.partial(
        pl.kernel,
        mesh=mesh,
        out_type=jax.ShapeDtypeStruct((n_seq, d_emb), jnp.float32),
        scratch_types=[
            pltpu.VMEM((_NBUF, _CHUNK_ROWS, d_emb), jnp.float32),
            pltpu.SemaphoreType.DMA((_NBUF,)),
            pltpu.SemaphoreType.DMA((_NBUF,)),
        ],
    )
    def copy_kernel(table_hbm, out_hbm, bufs, rsems, wsems):
        wid = lax.axis_index("s") * nc + lax.axis_index("c")
        base = wid * rows_per_w

        def start_read(i):
            return pltpu.async_copy(
                table_hbm.at[pl.ds(base + i * _CHUNK_ROWS, _CHUNK_ROWS)],
                bufs.at[i % _NBUF],
                rsems.at[i % _NBUF],
            )

        def start_write(i):
            return pltpu.async_copy(
                bufs.at[i % _NBUF],
                out_hbm.at[pl.ds(base + i * _CHUNK_ROWS, _CHUNK_ROWS)],
                wsems.at[i % _NBUF],
            )

        reads = [None] * n_chunks
        writes = [None] * n_chunks
        for i in range(min(_READ_AHEAD, n_chunks)):
            reads[i] = start_read(i)
        for i in range(n_chunks):
            nxt = i + _READ_AHEAD
            if nxt < n_chunks:
                if nxt >= _NBUF:
                    writes[nxt - _NBUF].wait()
                reads[nxt] = start_read(nxt)
            reads[i].wait()
            writes[i] = start_write(i)
        for i in range(max(0, n_chunks - _NBUF), n_chunks):
            writes[i].wait()

    return copy_kernel


def kernel(x, table):
    n_seq = x.shape[-1]
    return _make_copy(n_seq, table.shape[1])(table)
